# baseline (speedup 1.0000x reference)
import functools

import jax
import jax.numpy as jnp
from jax import lax
from jax.experimental import pallas as pl
from jax.experimental.pallas import tpu as pltpu
from jax.experimental.pallas import tpu_sc as plsc

NC = 2
NS = 16
L = 16
G = 5376


NBLK = 1280
FB = 8


def _make_front_body(T, NMASK):
    def body(x_ref, w_ref, att_ref, h_ref, aa_ref, mx_ref):
        w = w_ref[...]
        j = pl.program_id(1)
        lane = jax.lax.broadcasted_iota(jnp.int32, (2, NBLK), 1) + j * NBLK
        for t in range(T):
            xb = x_ref[0, t]
            hT = lax.dot_general(w, xb, (((0,), (1,)), ((), ())),
                                 preferred_element_type=jnp.float32)
            h_ref[0, t] = hT
            aa = jnp.dot(att_ref[...], hT)
            aa_ref[0, t] = aa
            cur = jnp.max(jnp.where(lane < NMASK, aa, -jnp.inf),
                          axis=1, keepdims=True)
            bc = jnp.broadcast_to(cur, (2, 16))

            @pl.when(j == 0)
            def _():
                mx_ref[0, t] = bc

            @pl.when(j > 0)
            def _():
                mx_ref[0, t] = jnp.maximum(mx_ref[0, t], bc)
    return body


def _densum_body(den_ref, out_ref):
    out_ref[0, :, 0, :] = jnp.sum(den_ref[0], axis=1) + 1e-16


def _final_body(num_ref, den_ref, bias_ref, out_ref):
    numT = jnp.transpose(num_ref[0, 0], (1, 0))
    den = jnp.transpose(den_ref[0, 0], (1, 0))
    out_ref[0, 0] = numT / den + bias_ref[...]


def _make_sc_kernel(BT, N, F, E1, EP):
    SH = max(1, (N - 1).bit_length())
    MK = (1 << SH) - 1
    FS = F // (NC * NS)
    EPT = EP // NS
    KA = EPT // G
    CB = EP // G
    GG = G // L
    NL = N // L

    mesh = plsc.VectorSubcoreMesh(core_axis_name="c", subcore_axis_name="s",
                                  num_cores=NC, num_subcores=NS)

    def body(sd_hbm, h_hbm, aa_hbm, mx_hbm, num_hbm, den_hbm,
             sd_v, p_v, sd2_v, p2_v, h_v, num_v, red_v,
             sem0, sem1, s_sd, s_p, s_zero):
        c = lax.axis_index("c")
        s = lax.axis_index("s")
        fbase = (c * NS + s) * FS
        OAS, OAD, ODEN = 0, N, 2 * N

        pltpu.sync_copy(sd_hbm.at[pl.ds(s * EPT, EPT)],
                        s_sd.at[pl.ds(s * EPT, EPT)])
        zz = jnp.zeros((L,), jnp.float32)

        def znum(i, _):
            num_v[pl.ds(i * L, L)] = zz
            return 0
        lax.fori_loop(0, FS * NL, znum, 0, unroll=8)

        @pl.when(s == 0)
        def _():
            pltpu.sync_copy(num_v, s_zero)
        plsc.subcore_barrier()

        def pass_body(bt, carry):
          with jax.named_scope("prep"):
            pltpu.sync_copy(aa_hbm.at[bt], h_v.at[pl.ds(OAS, 2 * N)])
            pltpu.sync_copy(mx_hbm.at[bt], red_v)
            mstar = red_v[pl.ds(0, L)] + red_v[pl.ds(L, L)]
            iot = lax.iota(jnp.int32, 16)
            pltpu.sync_copy(s_zero.at[pl.ds(0, N)], h_v.at[pl.ds(ODEN, N)])

          with jax.named_scope("phaseA"):
            def pa_chunk(k, _):
                off = s * EPT + k * G
                pltpu.sync_copy(s_sd.at[pl.ds(off, G)], sd_v)

                @plsc.parallel_loop(0, GG, unroll=2)
                def pa_group(g):
                    sd = sd_v[pl.ds(g * L, L)]
                    si = lax.shift_right_logical(sd, SH)
                    di = jnp.bitwise_and(sd, MK)
                    z = (plsc.load_gather(h_v, [si])
                         + plsc.load_gather(h_v, [di + OAD]))
                    e = jnp.where(z >= 0, z, 0.2 * z)
                    gid = off + g * L + iot
                    p = jnp.where(gid < E1, jnp.exp(e - mstar), 0.0)
                    p_v[pl.ds(g * L, L)] = p
                    plsc.addupdate_scatter(h_v, [di + ODEN], p)
                pltpu.sync_copy(p_v, s_p.at[pl.ds(off, G)])
                return 0
            lax.fori_loop(0, KA, pa_chunk, 0)

          with jax.named_scope("hstage"):
            @pl.when(c == 0)
            def _():
                pltpu.sync_copy(h_v.at[pl.ds(ODEN, N)], den_hbm.at[bt, s])

            pltpu.sync_copy(h_hbm.at[bt, pl.ds(fbase * N, FS * N)], h_v)
            pltpu.sync_copy(s_zero, num_v)

          with jax.named_scope("phaseB"):
            plsc.subcore_barrier()

            def issue_b(k, bufs):
                off = k * G
                sv, pv, sem = bufs
                pltpu.async_copy(s_sd.at[pl.ds(off, G)], sv, sem)
                pltpu.async_copy(s_p.at[pl.ds(off, G)], pv, sem)

            def wait_b(bufs):
                sv, pv, sem = bufs
                pltpu.make_async_copy(s_sd.at[pl.ds(0, G)], sv, sem).wait()
                pltpu.make_async_copy(s_p.at[pl.ds(0, G)], pv, sem).wait()

            ring = ((sd_v, p_v, sem0), (sd2_v, p2_v, sem1))
            issue_b(0, ring[0])

            def pb_outer(i, _):
                for b in range(2):
                    k = i * 2 + b
                    @pl.when(k + 1 < CB)
                    def _():
                        issue_b(k + 1, ring[1 - b])
                    wait_b(ring[b])
                    sv, pv, _sem = ring[b]

                    @plsc.parallel_loop(0, GG, unroll=2)
                    def pb_group(g):
                        sd = sv[pl.ds(g * L, L)]
                        si = lax.shift_right_logical(sd, SH)
                        di = jnp.bitwise_and(sd, MK)
                        pp = pv[pl.ds(g * L, L)]
                        for j in range(FS):
                            hv = plsc.load_gather(h_v, [si + j * N])
                            plsc.addupdate_scatter(num_v, [di + j * N], hv * pp)
                return 0
            lax.fori_loop(0, CB // 2, pb_outer, 0)

          with jax.named_scope("numout"):
            pltpu.sync_copy(num_v, num_hbm.at[bt, pl.ds(fbase * N, FS * N)])
            plsc.subcore_barrier()
          return carry
        lax.fori_loop(0, BT, pass_body, 0)

    return pl.kernel(
        body,
        out_type=[
            jax.ShapeDtypeStruct((BT, F * N), jnp.float32),
            jax.ShapeDtypeStruct((BT, NS, N), jnp.float32),
        ],
        mesh=mesh,
        compiler_params=pltpu.CompilerParams(needs_layout_passes=False,
                                             use_tc_tiling_on_sc=False),
        scratch_types=[
            pltpu.VMEM((G,), jnp.int32),
            pltpu.VMEM((G,), jnp.float32),
            pltpu.VMEM((G,), jnp.int32),
            pltpu.VMEM((G,), jnp.float32),
            pltpu.VMEM((FS * N,), jnp.float32),
            pltpu.VMEM((FS * N,), jnp.float32),
            pltpu.VMEM((2 * L,), jnp.float32),
            pltpu.SemaphoreType.DMA,
            pltpu.SemaphoreType.DMA,
            pltpu.VMEM_SHARED((EP,), jnp.int32),
            pltpu.VMEM_SHARED((EP,), jnp.float32),
            pltpu.VMEM_SHARED((FS * N,), jnp.float32),
        ],
    )


def kernel(x, edge_index, W, att_src, att_dst, bias):
    B, C, T, N = x.shape
    F = W.shape[1]
    E = edge_index.shape[1]
    BT = B * T
    E1 = E + N
    step = NS * G
    EP = ((E1 + step - 1) // step) * step

    loops = jnp.arange(N, dtype=edge_index.dtype)
    pad = jnp.zeros((EP - E1,), edge_index.dtype)
    src = jnp.concatenate([edge_index[0], loops, pad])
    dst = jnp.concatenate([edge_index[1], loops, pad])
    sd = jnp.bitwise_or(jnp.left_shift(src, max(1, (N - 1).bit_length())), dst)

    NB = (N + NBLK - 1) // NBLK
    hT, aa, mx = pl.pallas_call(
        _make_front_body(T, N),
        grid=(B, NB),
        in_specs=[
            pl.BlockSpec((1, T, NBLK, C), lambda b, j: (b, 0, j, 0)),
            pl.BlockSpec((C, F), lambda b, j: (0, 0)),
            pl.BlockSpec((2, F), lambda b, j: (0, 0)),
        ],
        out_specs=[
            pl.BlockSpec((1, T, F, NBLK), lambda b, j: (b, 0, 0, j)),
            pl.BlockSpec((1, T, 2, NBLK), lambda b, j: (b, 0, 0, j)),
            pl.BlockSpec((1, T, 2, 16), lambda b, j: (b, 0, 0, 0)),
        ],
        out_shape=[
            jax.ShapeDtypeStruct((B, T, F, N), jnp.float32),
            jax.ShapeDtypeStruct((B, T, 2, N), jnp.float32),
            jax.ShapeDtypeStruct((B, T, 2, 16), jnp.float32),
        ],
    )(jnp.transpose(x, (0, 2, 3, 1)), W, jnp.stack([att_src, att_dst]))

    num, den = _make_sc_kernel(BT, N, F, E1, EP)(
        sd,
        hT.reshape(BT, F * N),
        aa.reshape(BT, 2 * N),
        mx.reshape(BT, 32),
    )

    densum = pl.pallas_call(
        _densum_body,
        grid=(B,),
        in_specs=[pl.BlockSpec((1, T, NS, N), lambda b: (b, 0, 0, 0))],
        out_specs=pl.BlockSpec((1, T, 1, N), lambda b: (b, 0, 0, 0)),
        out_shape=jax.ShapeDtypeStruct((B, T, 1, N), jnp.float32),
    )(den.reshape(B, T, NS, N))

    out2 = pl.pallas_call(
        _final_body,
        grid=(B, T),
        in_specs=[
            pl.BlockSpec((1, 1, F, N), lambda b, t: (b, t, 0, 0)),
            pl.BlockSpec((1, 1, 1, N), lambda b, t: (b, t, 0, 0)),
            pl.BlockSpec((1, F), lambda b, t: (0, 0)),
        ],
        out_specs=pl.BlockSpec((1, 1, N, F), lambda b, t: (b, t, 0, 0)),
        out_shape=jax.ShapeDtypeStruct((B, T, N, F), jnp.float32),
    )(num.reshape(B, T, F, N), densum.reshape(B, T, 1, N), bias.reshape(1, F))

    return jnp.transpose(out2.reshape(B, T, N, F), (0, 3, 1, 2))


# speedup vs baseline: 1.0017x; 1.0017x over previous
import functools

import jax
import jax.numpy as jnp
from jax import lax
from jax.experimental import pallas as pl
from jax.experimental.pallas import tpu as pltpu
from jax.experimental.pallas import tpu_sc as plsc

NC = 2
NS = 16
L = 16
G = 5376


NBLK = 1280
FB = 8


def _make_front_body(T, NMASK):
    def body(x_ref, w_ref, att_ref, h_ref, aa_ref, mx_ref):
        w = w_ref[...]
        j = pl.program_id(1)
        lane = jax.lax.broadcasted_iota(jnp.int32, (2, NBLK), 1) + j * NBLK
        for t in range(T):
            xb = x_ref[0, t]
            hT = lax.dot_general(w, xb, (((0,), (1,)), ((), ())),
                                 preferred_element_type=jnp.float32)
            h_ref[0, t] = hT
            aa = jnp.dot(att_ref[...], hT)
            aa_ref[0, t] = aa
            cur = jnp.max(jnp.where(lane < NMASK, aa, -jnp.inf),
                          axis=1, keepdims=True)
            bc = jnp.broadcast_to(cur, (2, 16))

            @pl.when(j == 0)
            def _():
                mx_ref[0, t] = bc

            @pl.when(j > 0)
            def _():
                mx_ref[0, t] = jnp.maximum(mx_ref[0, t], bc)
    return body


def _densum_body(den_ref, out_ref):
    out_ref[0, :, 0, :] = jnp.sum(den_ref[0], axis=1) + 1e-16


def _final_body(num_ref, den_ref, bias_ref, out_ref):
    numT = jnp.transpose(num_ref[0, 0], (1, 0))
    den = jnp.transpose(den_ref[0, 0], (1, 0))
    out_ref[0, 0] = numT / den + bias_ref[...]


def _make_sc_kernel(BT, N, F, E1, EP):
    SH = max(1, (N - 1).bit_length())
    MK = (1 << SH) - 1
    FS = F // (NC * NS)
    EPT = EP // NS
    KA = EPT // G
    CB = EP // G
    GG = G // L
    NL = N // L

    mesh = plsc.VectorSubcoreMesh(core_axis_name="c", subcore_axis_name="s",
                                  num_cores=NC, num_subcores=NS)

    def body(sd_hbm, h_hbm, aa_hbm, mx_hbm, num_hbm, den_hbm,
             sd_v, p_v, sd2_v, p2_v, h_v, num_v, red_v,
             sem0, sem1, s_sd, s_p):
        c = lax.axis_index("c")
        s = lax.axis_index("s")
        fbase = (c * NS + s) * FS
        OAS, OAD, ODEN = 0, N, 2 * N

        pltpu.sync_copy(sd_hbm.at[pl.ds(s * EPT, EPT)],
                        s_sd.at[pl.ds(s * EPT, EPT)])
        zz = jnp.zeros((L,), jnp.float32)
        plsc.subcore_barrier()

        def pass_body(bt, carry):
          with jax.named_scope("prep"):
            pltpu.sync_copy(aa_hbm.at[bt], h_v.at[pl.ds(OAS, 2 * N)])
            pltpu.sync_copy(mx_hbm.at[bt], red_v)
            mstar = red_v[pl.ds(0, L)] + red_v[pl.ds(L, L)]
            iot = lax.iota(jnp.int32, 16)
            def zden(i, _):
                h_v[pl.ds(ODEN + i * L, L)] = zz
                return 0
            lax.fori_loop(0, NL, zden, 0, unroll=8)

          with jax.named_scope("phaseA"):
            def pa_chunk(k, _):
                off = s * EPT + k * G
                pltpu.sync_copy(s_sd.at[pl.ds(off, G)], sd_v)

                @plsc.parallel_loop(0, GG, unroll=2)
                def pa_group(g):
                    sd = sd_v[pl.ds(g * L, L)]
                    si = lax.shift_right_logical(sd, SH)
                    di = jnp.bitwise_and(sd, MK)
                    z = (plsc.load_gather(h_v, [si])
                         + plsc.load_gather(h_v, [di + OAD]))
                    e = jnp.where(z >= 0, z, 0.2 * z)
                    gid = off + g * L + iot
                    p = jnp.where(gid < E1, jnp.exp(e - mstar), 0.0)
                    p_v[pl.ds(g * L, L)] = p
                    plsc.addupdate_scatter(h_v, [di + ODEN], p)
                pltpu.sync_copy(p_v, s_p.at[pl.ds(off, G)])
                return 0
            lax.fori_loop(0, KA, pa_chunk, 0)

          with jax.named_scope("hstage"):
            @pl.when(c == 0)
            def _():
                pltpu.sync_copy(h_v.at[pl.ds(ODEN, N)], den_hbm.at[bt, s])

            pltpu.sync_copy(h_hbm.at[bt, pl.ds(fbase * N, FS * N)], h_v)
            def znum(i, _):
                num_v[pl.ds(i * L, L)] = zz
                return 0
            lax.fori_loop(0, FS * NL, znum, 0, unroll=8)

          with jax.named_scope("phaseB"):
            plsc.subcore_barrier()

            def issue_b(k, bufs):
                off = k * G
                sv, pv, sem = bufs
                pltpu.async_copy(s_sd.at[pl.ds(off, G)], sv, sem)
                pltpu.async_copy(s_p.at[pl.ds(off, G)], pv, sem)

            def wait_b(bufs):
                sv, pv, sem = bufs
                pltpu.make_async_copy(s_sd.at[pl.ds(0, G)], sv, sem).wait()
                pltpu.make_async_copy(s_p.at[pl.ds(0, G)], pv, sem).wait()

            ring = ((sd_v, p_v, sem0), (sd2_v, p2_v, sem1))
            issue_b(0, ring[0])

            def pb_outer(i, _):
                for b in range(2):
                    k = i * 2 + b
                    @pl.when(k + 1 < CB)
                    def _():
                        issue_b(k + 1, ring[1 - b])
                    wait_b(ring[b])
                    sv, pv, _sem = ring[b]

                    @plsc.parallel_loop(0, GG, unroll=2)
                    def pb_group(g):
                        sd = sv[pl.ds(g * L, L)]
                        si = lax.shift_right_logical(sd, SH)
                        di = jnp.bitwise_and(sd, MK)
                        pp = pv[pl.ds(g * L, L)]
                        for j in range(FS):
                            hv = plsc.load_gather(h_v, [si + j * N])
                            plsc.addupdate_scatter(num_v, [di + j * N], hv * pp)
                return 0
            lax.fori_loop(0, CB // 2, pb_outer, 0)

          with jax.named_scope("numout"):
            pltpu.sync_copy(num_v, num_hbm.at[bt, pl.ds(fbase * N, FS * N)])
            plsc.subcore_barrier()
          return carry
        lax.fori_loop(0, BT, pass_body, 0)

    return pl.kernel(
        body,
        out_type=[
            jax.ShapeDtypeStruct((BT, F * N), jnp.float32),
            jax.ShapeDtypeStruct((BT, NS, N), jnp.float32),
        ],
        mesh=mesh,
        compiler_params=pltpu.CompilerParams(needs_layout_passes=False,
                                             use_tc_tiling_on_sc=False),
        scratch_types=[
            pltpu.VMEM((G,), jnp.int32),
            pltpu.VMEM((G,), jnp.float32),
            pltpu.VMEM((G,), jnp.int32),
            pltpu.VMEM((G,), jnp.float32),
            pltpu.VMEM((FS * N,), jnp.float32),
            pltpu.VMEM((FS * N,), jnp.float32),
            pltpu.VMEM((2 * L,), jnp.float32),
            pltpu.SemaphoreType.DMA,
            pltpu.SemaphoreType.DMA,
            pltpu.VMEM_SHARED((EP,), jnp.int32),
            pltpu.VMEM_SHARED((EP,), jnp.float32),
        ],
    )


def kernel(x, edge_index, W, att_src, att_dst, bias):
    B, C, T, N = x.shape
    F = W.shape[1]
    E = edge_index.shape[1]
    BT = B * T
    E1 = E + N
    step = NS * G
    EP = ((E1 + step - 1) // step) * step

    loops = jnp.arange(N, dtype=edge_index.dtype)
    pad = jnp.zeros((EP - E1,), edge_index.dtype)
    src = jnp.concatenate([edge_index[0], loops, pad])
    dst = jnp.concatenate([edge_index[1], loops, pad])
    sd = jnp.bitwise_or(jnp.left_shift(src, max(1, (N - 1).bit_length())), dst)

    NB = (N + NBLK - 1) // NBLK
    hT, aa, mx = pl.pallas_call(
        _make_front_body(T, N),
        grid=(B, NB),
        in_specs=[
            pl.BlockSpec((1, T, NBLK, C), lambda b, j: (b, 0, j, 0)),
            pl.BlockSpec((C, F), lambda b, j: (0, 0)),
            pl.BlockSpec((2, F), lambda b, j: (0, 0)),
        ],
        out_specs=[
            pl.BlockSpec((1, T, F, NBLK), lambda b, j: (b, 0, 0, j)),
            pl.BlockSpec((1, T, 2, NBLK), lambda b, j: (b, 0, 0, j)),
            pl.BlockSpec((1, T, 2, 16), lambda b, j: (b, 0, 0, 0)),
        ],
        out_shape=[
            jax.ShapeDtypeStruct((B, T, F, N), jnp.float32),
            jax.ShapeDtypeStruct((B, T, 2, N), jnp.float32),
            jax.ShapeDtypeStruct((B, T, 2, 16), jnp.float32),
        ],
    )(jnp.transpose(x, (0, 2, 3, 1)), W, jnp.stack([att_src, att_dst]))

    num, den = _make_sc_kernel(BT, N, F, E1, EP)(
        sd,
        hT.reshape(BT, F * N),
        aa.reshape(BT, 2 * N),
        mx.reshape(BT, 32),
    )

    densum = pl.pallas_call(
        _densum_body,
        grid=(B,),
        in_specs=[pl.BlockSpec((1, T, NS, N), lambda b: (b, 0, 0, 0))],
        out_specs=pl.BlockSpec((1, T, 1, N), lambda b: (b, 0, 0, 0)),
        out_shape=jax.ShapeDtypeStruct((B, T, 1, N), jnp.float32),
    )(den.reshape(B, T, NS, N))

    out2 = pl.pallas_call(
        _final_body,
        grid=(B, T),
        in_specs=[
            pl.BlockSpec((1, 1, F, N), lambda b, t: (b, t, 0, 0)),
            pl.BlockSpec((1, 1, 1, N), lambda b, t: (b, t, 0, 0)),
            pl.BlockSpec((1, F), lambda b, t: (0, 0)),
        ],
        out_specs=pl.BlockSpec((1, 1, N, F), lambda b, t: (b, t, 0, 0)),
        out_shape=jax.ShapeDtypeStruct((B, T, N, F), jnp.float32),
    )(num.reshape(B, T, F, N), densum.reshape(B, T, 1, N), bias.reshape(1, F))

    return jnp.transpose(out2.reshape(B, T, N, F), (0, 3, 1, 2))


# speedup vs baseline: 1.0083x; 1.0066x over previous
import functools

import jax
import jax.numpy as jnp
from jax import lax
from jax.experimental import pallas as pl
from jax.experimental.pallas import tpu as pltpu
from jax.experimental.pallas import tpu_sc as plsc

NC = 2
NS = 16
L = 16
G = 5376


NBLK = 1280
FB = 8


def _make_front_body(T):
    def body(x_ref, w_ref, att_ref, h_ref, aa_ref):
        w = w_ref[...]
        for t in range(T):
            xb = x_ref[0, t]
            hT = lax.dot_general(w, xb, (((0,), (1,)), ((), ())),
                                 preferred_element_type=jnp.float32)
            h_ref[0, t] = hT
            aa_ref[0, t] = jnp.dot(att_ref[...], hT)
    return body


def _densum_body(den_ref, out_ref):
    out_ref[0, :, 0, :] = jnp.sum(den_ref[0], axis=1) + 1e-16


def _final_body(num_ref, den_ref, bias_ref, out_ref):
    numT = jnp.transpose(num_ref[0, 0], (1, 0))
    den = jnp.transpose(den_ref[0, 0], (1, 0))
    out_ref[0, 0] = numT / den + bias_ref[...]


def _make_sc_kernel(BT, N, F, E1, EP):
    SH = max(1, (N - 1).bit_length())
    MK = (1 << SH) - 1
    FS = F // (NC * NS)
    EPT = EP // NS
    KA = EPT // G
    CB = EP // G
    GG = G // L
    NL = N // L

    mesh = plsc.VectorSubcoreMesh(core_axis_name="c", subcore_axis_name="s",
                                  num_cores=NC, num_subcores=NS)

    def body(sd_hbm, h_hbm, aa_hbm, num_hbm, den_hbm,
             sd_v, p_v, sd2_v, p2_v, h_v, num_v, red_v,
             sem0, sem1, s_sd, s_p):
        c = lax.axis_index("c")
        s = lax.axis_index("s")
        fbase = (c * NS + s) * FS
        OAS, OAD, ODEN = 0, N, 2 * N

        pltpu.sync_copy(sd_hbm.at[pl.ds(s * EPT, EPT)],
                        s_sd.at[pl.ds(s * EPT, EPT)])
        zz = jnp.zeros((L,), jnp.float32)
        plsc.subcore_barrier()

        def pass_body(bt, carry):
          with jax.named_scope("prep"):
            pltpu.sync_copy(aa_hbm.at[bt], h_v.at[pl.ds(OAS, 2 * N)])

            def maxstep(i, m):
                ms, md = m
                return (jnp.maximum(ms, h_v[pl.ds(OAS + i * L, L)]),
                        jnp.maximum(md, h_v[pl.ds(OAD + i * L, L)]))
            neg = jnp.full((L,), -jnp.inf, jnp.float32)
            ms, md = lax.fori_loop(0, NL, maxstep, (neg, neg), unroll=8)

            iot = lax.iota(jnp.int32, 16)
            def lane_allmax(v):
                for k in (1, 2, 4, 8):
                    red_v[...] = v
                    v = jnp.maximum(v, plsc.load_gather(red_v, [iot ^ k]))
                return v
            mstar = lane_allmax(ms) + lane_allmax(md)

            def zden(i, _):
                h_v[pl.ds(ODEN + i * L, L)] = zz
                return 0
            lax.fori_loop(0, NL, zden, 0, unroll=8)

          with jax.named_scope("phaseA"):
            def pa_chunk(k, _):
                off = s * EPT + k * G
                pltpu.sync_copy(s_sd.at[pl.ds(off, G)], sd_v)

                @plsc.parallel_loop(0, GG, unroll=2)
                def pa_group(g):
                    sd = sd_v[pl.ds(g * L, L)]
                    si = lax.shift_right_logical(sd, SH)
                    di = jnp.bitwise_and(sd, MK)
                    z = (plsc.load_gather(h_v, [si])
                         + plsc.load_gather(h_v, [di + OAD]))
                    e = jnp.where(z >= 0, z, 0.2 * z)
                    gid = off + g * L + iot
                    p = jnp.where(gid < E1, jnp.exp(e - mstar), 0.0)
                    p_v[pl.ds(g * L, L)] = p
                    plsc.addupdate_scatter(h_v, [di + ODEN], p)
                pltpu.sync_copy(p_v, s_p.at[pl.ds(off, G)])
                return 0
            lax.fori_loop(0, KA, pa_chunk, 0)

          with jax.named_scope("hstage"):
            @pl.when(c == 0)
            def _():
                pltpu.sync_copy(h_v.at[pl.ds(ODEN, N)], den_hbm.at[bt, s])

            pltpu.sync_copy(h_hbm.at[bt, pl.ds(fbase * N, FS * N)], h_v)
            def znum(i, _):
                num_v[pl.ds(i * L, L)] = zz
                return 0
            lax.fori_loop(0, FS * NL, znum, 0, unroll=8)

          with jax.named_scope("phaseB"):
            plsc.subcore_barrier()

            def issue_b(k, bufs):
                off = k * G
                sv, pv, sem = bufs
                pltpu.async_copy(s_sd.at[pl.ds(off, G)], sv, sem)
                pltpu.async_copy(s_p.at[pl.ds(off, G)], pv, sem)

            def wait_b(bufs):
                sv, pv, sem = bufs
                pltpu.make_async_copy(s_sd.at[pl.ds(0, G)], sv, sem).wait()
                pltpu.make_async_copy(s_p.at[pl.ds(0, G)], pv, sem).wait()

            ring = ((sd_v, p_v, sem0), (sd2_v, p2_v, sem1))
            issue_b(0, ring[0])

            def pb_outer(i, _):
                for b in range(2):
                    k = i * 2 + b
                    @pl.when(k + 1 < CB)
                    def _():
                        issue_b(k + 1, ring[1 - b])
                    wait_b(ring[b])
                    sv, pv, _sem = ring[b]

                    @plsc.parallel_loop(0, GG, unroll=2)
                    def pb_group(g):
                        sd = sv[pl.ds(g * L, L)]
                        si = lax.shift_right_logical(sd, SH)
                        di = jnp.bitwise_and(sd, MK)
                        pp = pv[pl.ds(g * L, L)]
                        for j in range(FS):
                            hv = plsc.load_gather(h_v, [si + j * N])
                            plsc.addupdate_scatter(num_v, [di + j * N], hv * pp)
                return 0
            lax.fori_loop(0, CB // 2, pb_outer, 0)

          with jax.named_scope("numout"):
            pltpu.sync_copy(num_v, num_hbm.at[bt, pl.ds(fbase * N, FS * N)])
            plsc.subcore_barrier()
          return carry
        lax.fori_loop(0, BT, pass_body, 0)

    return pl.kernel(
        body,
        out_type=[
            jax.ShapeDtypeStruct((BT, F * N), jnp.float32),
            jax.ShapeDtypeStruct((BT, NS, N), jnp.float32),
        ],
        mesh=mesh,
        compiler_params=pltpu.CompilerParams(needs_layout_passes=False,
                                             use_tc_tiling_on_sc=False),
        scratch_types=[
            pltpu.VMEM((G,), jnp.int32),
            pltpu.VMEM((G,), jnp.float32),
            pltpu.VMEM((G,), jnp.int32),
            pltpu.VMEM((G,), jnp.float32),
            pltpu.VMEM((FS * N,), jnp.float32),
            pltpu.VMEM((FS * N,), jnp.float32),
            pltpu.VMEM((L,), jnp.float32),
            pltpu.SemaphoreType.DMA,
            pltpu.SemaphoreType.DMA,
            pltpu.VMEM_SHARED((EP,), jnp.int32),
            pltpu.VMEM_SHARED((EP,), jnp.float32),
        ],
    )


def kernel(x, edge_index, W, att_src, att_dst, bias):
    B, C, T, N = x.shape
    F = W.shape[1]
    E = edge_index.shape[1]
    BT = B * T
    E1 = E + N
    step = NS * G
    EP = ((E1 + step - 1) // step) * step

    loops = jnp.arange(N, dtype=edge_index.dtype)
    pad = jnp.zeros((EP - E1,), edge_index.dtype)
    src = jnp.concatenate([edge_index[0], loops, pad])
    dst = jnp.concatenate([edge_index[1], loops, pad])
    sd = jnp.bitwise_or(jnp.left_shift(src, max(1, (N - 1).bit_length())), dst)

    NB = (N + NBLK - 1) // NBLK
    hT, aa = pl.pallas_call(
        _make_front_body(T),
        grid=(B, NB),
        in_specs=[
            pl.BlockSpec((1, T, NBLK, C), lambda b, j: (b, 0, j, 0)),
            pl.BlockSpec((C, F), lambda b, j: (0, 0)),
            pl.BlockSpec((2, F), lambda b, j: (0, 0)),
        ],
        out_specs=[
            pl.BlockSpec((1, T, F, NBLK), lambda b, j: (b, 0, 0, j)),
            pl.BlockSpec((1, T, 2, NBLK), lambda b, j: (b, 0, 0, j)),
        ],
        out_shape=[
            jax.ShapeDtypeStruct((B, T, F, N), jnp.float32),
            jax.ShapeDtypeStruct((B, T, 2, N), jnp.float32),
        ],
    )(jnp.transpose(x, (0, 2, 3, 1)), W, jnp.stack([att_src, att_dst]))

    num, den = _make_sc_kernel(BT, N, F, E1, EP)(
        sd,
        hT.reshape(BT, F * N),
        aa.reshape(BT, 2 * N),
    )

    densum = pl.pallas_call(
        _densum_body,
        grid=(B,),
        in_specs=[pl.BlockSpec((1, T, NS, N), lambda b: (b, 0, 0, 0))],
        out_specs=pl.BlockSpec((1, T, 1, N), lambda b: (b, 0, 0, 0)),
        out_shape=jax.ShapeDtypeStruct((B, T, 1, N), jnp.float32),
    )(den.reshape(B, T, NS, N))

    out2 = pl.pallas_call(
        _final_body,
        grid=(B, T),
        in_specs=[
            pl.BlockSpec((1, 1, F, N), lambda b, t: (b, t, 0, 0)),
            pl.BlockSpec((1, 1, 1, N), lambda b, t: (b, t, 0, 0)),
            pl.BlockSpec((1, F), lambda b, t: (0, 0)),
        ],
        out_specs=pl.BlockSpec((1, 1, N, F), lambda b, t: (b, t, 0, 0)),
        out_shape=jax.ShapeDtypeStruct((B, T, N, F), jnp.float32),
    )(num.reshape(B, T, F, N), densum.reshape(B, T, 1, N), bias.reshape(1, F))

    return jnp.transpose(out2.reshape(B, T, N, F), (0, 3, 1, 2))
